# baseline (speedup 1.0000x reference)
import dataclasses
import functools

import jax
import jax.numpy as jnp
from jax import lax
from jax.experimental import pallas as pl
from jax.experimental.pallas import tpu as pltpu
from jax.experimental.pallas import tpu_sc as plsc

_NC = 2
_NS = 16
_NW = _NC * _NS
_L = 16
_BLK = 80
_NBT = 64
_CH = _BLK * _NBT
_BLKC = 80


def _rsqrt_scalar(x):
    i = lax.bitcast_convert_type(x, jnp.int32)
    i = jnp.int32(0x5F3759DF) - lax.shift_right_logical(i, 1)
    y = lax.bitcast_convert_type(i, jnp.float32)
    for _ in range(3):
        y = y * (1.5 - 0.5 * x * y * y)
    return y


def _prep_body(nproj, n_ref, ws_ref, wr_ref, e_ref, w0_ref, g_ref, wg_ref,
               b_ref, os_ref, or_ref, oe_ref):
    c = jnp.dot(g_ref[...], wg_ref[...], preferred_element_type=jnp.float32)
    oe_ref[...] = (
        jnp.dot(e_ref[...], w0_ref[...], preferred_element_type=jnp.float32)
        + c + b_ref[...])

    @pl.when(pl.program_id(0) < nproj)
    def _():
        x = n_ref[...]
        os_ref[...] = jnp.dot(x, ws_ref[...],
                              preferred_element_type=jnp.float32)
        or_ref[...] = jnp.dot(x, wr_ref[...],
                              preferred_element_type=jnp.float32)


def _node_body(n_ref, s_ref, r_ref, w1_ref, w2_ref, w3_ref, g_ref, wg_ref,
               b_ref, gam_ref, bet_ref, o_ref):
    y = (jnp.dot(n_ref[...], w1_ref[...], preferred_element_type=jnp.float32)
         + jnp.dot(s_ref[0], w2_ref[...], preferred_element_type=jnp.float32)
         + jnp.dot(r_ref[0], w3_ref[...], preferred_element_type=jnp.float32)
         + jnp.dot(g_ref[...], wg_ref[...], preferred_element_type=jnp.float32)
         + b_ref[...])
    mu = jnp.mean(y, axis=-1, keepdims=True)
    var = jnp.mean(y * y, axis=-1, keepdims=True) - mu * mu
    yn = (y - mu) * lax.rsqrt(var + 1e-5)
    o_ref[...] = jnp.maximum(yn * gam_ref[...] + bet_ref[...], 0.0)


def _edge_sc_body(ps_hbm, pr_hbm, ep_hbm, snd_hbm, rcv_hbm, ge_hbm, be_hbm,
                  out_hbm, sidx_all, ridx_all, gs2, gr2, ep2, yb2, gam, bet,
                  ss0, sr0, se0, sw0, ss1, sr1, se1, sw1):
    E, H = out_hbm.shape
    nv = H // _L
    wid = lax.axis_index("s") * _NC + lax.axis_index("c")
    base_blk = wid * _NBT
    last = jnp.minimum(_NBT, E // _BLK - base_blk) - 1
    sems = {0: (ss0, sr0, se0, sw0), 1: (ss1, sr1, se1, sw1)}
    bufs = {0: (gs2.at[0], gr2.at[0], ep2.at[0], yb2.at[0]),
            1: (gs2.at[1], gr2.at[1], ep2.at[1], yb2.at[1])}

    pltpu.sync_copy(ge_hbm, gam)
    pltpu.sync_copy(be_hbm, bet)
    pltpu.sync_copy(snd_hbm.at[pl.ds(wid * _CH, _CH)], sidx_all)
    pltpu.sync_copy(rcv_hbm.at[pl.ds(wid * _CH, _CH)], ridx_all)

    def _gathers(p, b):
        bc = jnp.minimum(b, last)
        off = (base_blk + bc) * _BLK
        gs_b, gr_b, ep_b, _ = bufs[p]
        sm = sems[p]
        return (
            pltpu.make_async_copy(
                ps_hbm.at[sidx_all.at[pl.ds(bc * _BLK, _BLK)]], gs_b, sm[0]),
            pltpu.make_async_copy(
                pr_hbm.at[ridx_all.at[pl.ds(bc * _BLK, _BLK)]], gr_b, sm[1]),
            pltpu.make_async_copy(ep_hbm.at[pl.ds(off, _BLK)], ep_b, sm[2]),
        )

    def _wb(p, b):
        bc = jnp.minimum(b, last)
        off = (base_blk + bc) * _BLK
        return pltpu.make_async_copy(bufs[p][3], out_hbm.at[pl.ds(off, _BLK)],
                                     sems[p][3])

    def _start_g(p, b):
        for cp in _gathers(p, b):
            cp.start()

    def _wait_g(p, b):
        for cp in _gathers(p, b):
            cp.wait()

    def _compute(p):
        @plsc.parallel_loop(0, _BLK, 1, unroll=16)
        def _edge(e):
            ys = [ep2[p, e, pl.ds(_L * j, _L)] + gs2[p, e, pl.ds(_L * j, _L)]
                  + gr2[p, e, pl.ds(_L * j, _L)] for j in range(nv)]
            acc = ys[0]
            for j in range(1, nv):
                acc = acc + ys[j]
            s1 = jnp.sum(acc)
            acc2 = ys[0] * ys[0]
            for j in range(1, nv):
                acc2 = acc2 + ys[j] * ys[j]
            s2 = jnp.sum(acc2)
            mu = s1 * (1.0 / H)
            var = s2 * (1.0 / H) - mu * mu
            inv = _rsqrt_scalar(var + 1e-5)
            for j in range(nv):
                sl = pl.ds(_L * j, _L)
                o = (ys[j] - mu) * inv * gam[sl] + bet[sl]
                yb2[p, e, sl] = jnp.maximum(o, 0.0)

    nh = (last + 1) // 2
    _start_g(0, 0)
    _start_g(1, 1)
    _wait_g(0, 0)
    _compute(0)
    _wb(0, 0).start()
    _start_g(0, 2)
    _wait_g(1, 1)
    _compute(1)
    _wb(1, 1).start()
    _start_g(1, 3)

    @pl.loop(1, nh - 1)
    def _h(h):
        b0 = 2 * h
        b1 = b0 + 1
        _wait_g(0, b0)
        _wb(0, b0 - 2).wait()
        _compute(0)
        _wb(0, b0).start()
        _start_g(0, b0 + 2)
        _wait_g(1, b1)
        _wb(1, b1 - 2).wait()
        _compute(1)
        _wb(1, b1).start()
        _start_g(1, b1 + 2)

    b0 = 2 * (nh - 1)
    b1 = b0 + 1
    _wait_g(0, b0)
    _wb(0, b0 - 2).wait()
    _compute(0)
    _wb(0, b0).start()
    _wait_g(1, b1)
    _wb(1, b1 - 2).wait()
    _compute(1)
    _wb(1, b1).start()
    _wb(0, b0).wait()
    _wb(1, b1).wait()


def _agg_sc_body(rows_hbm, srcat_hbm, out_hbm, idx2, rows2, zb, acc,
                 si0, si1, sr0, sr1):
    E, H = rows_hbm.shape
    N = acc.shape[0]
    c = lax.axis_index("c")
    s = lax.axis_index("s")
    nzb = zb.shape[0]
    nzblk = N // nzb
    rounds = (nzblk + _NS - 1) // _NS

    @pl.loop(0, nzb)
    def _zrow(r):
        for j in range(H // _L):
            zb[r, pl.ds(_L * j, _L)] = jnp.zeros((_L,), jnp.float32)

    for q in range(rounds):
        g = s + q * _NS

        @pl.when(g < nzblk)
        def _():
            pltpu.sync_copy(zb, acc.at[pl.ds(g * nzb, nzb)])

    plsc.subcore_barrier()

    ept = E // _NS
    nblk = ept // _BLKC
    lastb = nblk - 1
    isems = {0: si0, 1: si1}
    rsems = {0: sr0, 1: sr1}

    def _in(p, b):
        bc = jnp.minimum(b, lastb)
        off = s * ept + bc * _BLKC
        return (
            pltpu.make_async_copy(srcat_hbm.at[pl.ds(c * E + off, _BLKC)],
                                  idx2.at[p], isems[p]),
            pltpu.make_async_copy(rows_hbm.at[pl.ds(off, _BLKC)],
                                  rows2.at[p], rsems[p]),
        )

    def _start_in(p, b):
        for cp in _in(p, b):
            cp.start()

    def _wait_in(p, b):
        for cp in _in(p, b):
            cp.wait()

    def _scatter(p):
        pltpu.sync_copy(rows2.at[p], acc.at[idx2.at[p]], add=True)

    _start_in(0, 0)
    _start_in(1, 1)

    @pl.loop(0, nblk // 2)
    def _h(h):
        b0 = 2 * h
        b1 = b0 + 1
        _wait_in(0, b0)
        _scatter(0)
        _start_in(0, b0 + 2)
        _wait_in(1, b1)
        _scatter(1)
        _start_in(1, b1 + 2)

    _wait_in(0, lastb)
    _scatter(0)
    _wait_in(1, lastb + 1)

    plsc.subcore_barrier()
    for q in range(rounds):
        g = s + q * _NS

        @pl.when(g < nzblk)
        def _():
            sl = pl.ds(g * nzb, nzb)
            pltpu.sync_copy(acc.at[sl], out_hbm.at[c, sl])


def kernel(nodes, edges, receivers, senders, globals_, n_node, n_edge,
           W_e, b_e, g_e, beta_e, W_n, b_n, g_n, beta_n):
    N, Dn = nodes.shape
    E, De = edges.shape
    Dg = globals_.shape[1]
    H = W_e.shape[1]

    W_e0 = W_e[:De]
    W_es = W_e[De:De + Dn]
    W_er = W_e[De + Dn:De + 2 * Dn]
    W_eg = W_e[De + 2 * Dn:]
    W_n1 = W_n[:Dn]
    W_n2 = W_n[Dn:Dn + H]
    W_n3 = W_n[Dn + H:Dn + 2 * H]
    W_ng = W_n[Dn + 2 * H:]

    snd = senders.astype(jnp.int32)
    rcv = receivers.astype(jnp.int32)
    srcat = jnp.concatenate([snd, rcv])
    pad = _NW * _CH - E
    snd_p = jnp.pad(snd, (0, pad))
    rcv_p = jnp.pad(rcv, (0, pad))

    bn = 1000
    be = 4000

    nproj = N // bn
    P_s, P_r, E_pre = pl.pallas_call(
        functools.partial(_prep_body, nproj),
        grid=(E // be,),
        in_specs=[
            pl.BlockSpec((bn, Dn), lambda i: (jnp.minimum(i, nproj - 1), 0)),
            pl.BlockSpec((Dn, H), lambda i: (0, 0)),
            pl.BlockSpec((Dn, H), lambda i: (0, 0)),
            pl.BlockSpec((be, De), lambda i: (i, 0)),
            pl.BlockSpec((De, H), lambda i: (0, 0)),
            pl.BlockSpec((1, Dg), lambda i: (0, 0)),
            pl.BlockSpec((Dg, H), lambda i: (0, 0)),
            pl.BlockSpec((1, H), lambda i: (0, 0)),
        ],
        out_specs=[
            pl.BlockSpec((bn, H), lambda i: (jnp.minimum(i, nproj - 1), 0)),
            pl.BlockSpec((bn, H), lambda i: (jnp.minimum(i, nproj - 1), 0)),
            pl.BlockSpec((be, H), lambda i: (i, 0)),
        ],
        out_shape=[
            jax.ShapeDtypeStruct((N, H), jnp.float32),
            jax.ShapeDtypeStruct((N, H), jnp.float32),
            jax.ShapeDtypeStruct((E, H), jnp.float32),
        ],
    )(nodes, W_es, W_er, edges, W_e0, globals_, W_eg, b_e.reshape(1, H))

    mesh = plsc.VectorSubcoreMesh(core_axis_name="c", subcore_axis_name="s")
    sc_params = pltpu.CompilerParams()
    if "needs_layout_passes" in getattr(
            pltpu.CompilerParams, "__dataclass_fields__", {}):
        sc_params = dataclasses.replace(sc_params, needs_layout_passes=False)

    edge_sc = pl.kernel(
        _edge_sc_body,
        out_type=jax.ShapeDtypeStruct((E, H), jnp.float32),
        mesh=mesh,
        scratch_types=[
            pltpu.VMEM((_CH,), jnp.int32),
            pltpu.VMEM((_CH,), jnp.int32),
            pltpu.VMEM((2, _BLK, H), jnp.float32),
            pltpu.VMEM((2, _BLK, H), jnp.float32),
            pltpu.VMEM((2, _BLK, H), jnp.float32),
            pltpu.VMEM((2, _BLK, H), jnp.float32),
            pltpu.VMEM((H,), jnp.float32),
            pltpu.VMEM((H,), jnp.float32),
        ] + [pltpu.SemaphoreType.DMA] * 8,
        compiler_params=sc_params,
    )
    new_edges = edge_sc(P_s, P_r, E_pre, snd_p, rcv_p, g_e, beta_e)

    agg_sc = pl.kernel(
        _agg_sc_body,
        out_type=jax.ShapeDtypeStruct((2, N, H), jnp.float32),
        mesh=mesh,
        scratch_types=[
            pltpu.VMEM((2, _BLKC), jnp.int32),
            pltpu.VMEM((2, _BLKC, H), jnp.float32),
            pltpu.VMEM((200, H), jnp.float32),
            pltpu.VMEM_SHARED((N, H), jnp.float32),
        ] + [pltpu.SemaphoreType.DMA] * 4,
    )
    aggs = agg_sc(new_edges, srcat)

    new_nodes = pl.pallas_call(
        _node_body,
        grid=(N // bn,),
        in_specs=[
            pl.BlockSpec((bn, Dn), lambda i: (i, 0)),
            pl.BlockSpec((1, bn, H), lambda i: (0, i, 0)),
            pl.BlockSpec((1, bn, H), lambda i: (1, i, 0)),
            pl.BlockSpec((Dn, H), lambda i: (0, 0)),
            pl.BlockSpec((H, H), lambda i: (0, 0)),
            pl.BlockSpec((H, H), lambda i: (0, 0)),
            pl.BlockSpec((1, Dg), lambda i: (0, 0)),
            pl.BlockSpec((Dg, H), lambda i: (0, 0)),
            pl.BlockSpec((1, H), lambda i: (0, 0)),
            pl.BlockSpec((1, H), lambda i: (0, 0)),
            pl.BlockSpec((1, H), lambda i: (0, 0)),
        ],
        out_specs=pl.BlockSpec((bn, H), lambda i: (i, 0)),
        out_shape=jax.ShapeDtypeStruct((N, H), jnp.float32),
    )(nodes, aggs, aggs, W_n1, W_n2, W_n3, globals_, W_ng,
      b_n.reshape(1, H), g_n.reshape(1, H), beta_n.reshape(1, H))

    return (new_nodes, new_edges)


# speedup vs baseline: 1.0736x; 1.0736x over previous
import dataclasses
import functools

import jax
import jax.numpy as jnp
from jax import lax
from jax.experimental import pallas as pl
from jax.experimental.pallas import tpu as pltpu
from jax.experimental.pallas import tpu_sc as plsc

_NC = 2
_NS = 16
_NW = _NC * _NS
_L = 16
_BLK = 80
_NBT = 64
_CH = _BLK * _NBT
_BLKC = 80


def _rsqrt_scalar(x):
    i = lax.bitcast_convert_type(x, jnp.int32)
    i = jnp.int32(0x5F3759DF) - lax.shift_right_logical(i, 1)
    y = lax.bitcast_convert_type(i, jnp.float32)
    for _ in range(3):
        y = y * (1.5 - 0.5 * x * y * y)
    return y


def _prep_body(nproj, n_ref, ws_ref, wr_ref, e_ref, w0_ref, g_ref, wg_ref,
               b_ref, os_ref, or_ref, oe_ref):
    c = jnp.dot(g_ref[...], wg_ref[...], preferred_element_type=jnp.float32)
    oe_ref[...] = (
        jnp.dot(e_ref[...], w0_ref[...], preferred_element_type=jnp.float32)
        + c + b_ref[...])

    @pl.when(pl.program_id(0) < nproj)
    def _():
        x = n_ref[...]
        os_ref[...] = jnp.dot(x, ws_ref[...],
                              preferred_element_type=jnp.float32)
        or_ref[...] = jnp.dot(x, wr_ref[...],
                              preferred_element_type=jnp.float32)


def _node_body(n_ref, s_ref, r_ref, w1_ref, w2_ref, w3_ref, g_ref, wg_ref,
               b_ref, gam_ref, bet_ref, o_ref):
    y = (jnp.dot(n_ref[...], w1_ref[...], preferred_element_type=jnp.float32)
         + jnp.dot(s_ref[0], w2_ref[...], preferred_element_type=jnp.float32)
         + jnp.dot(r_ref[0], w3_ref[...], preferred_element_type=jnp.float32)
         + jnp.dot(g_ref[...], wg_ref[...], preferred_element_type=jnp.float32)
         + b_ref[...])
    mu = jnp.mean(y, axis=-1, keepdims=True)
    var = jnp.mean(y * y, axis=-1, keepdims=True) - mu * mu
    yn = (y - mu) * lax.rsqrt(var + 1e-5)
    o_ref[...] = jnp.maximum(yn * gam_ref[...] + bet_ref[...], 0.0)


def _edge_sc_body(ps_hbm, pr_hbm, ep_hbm, snd_hbm, rcv_hbm, ge_hbm, be_hbm,
                  out_hbm, sidx_all, ridx_all, gs2, gr2, ep2, yb2, gam, bet,
                  ss0, sr0, se0, sw0, ss1, sr1, se1, sw1):
    E, H = out_hbm.shape
    nv = H // _L
    wid = lax.axis_index("s") * _NC + lax.axis_index("c")
    base_blk = wid * _NBT
    last = jnp.minimum(_NBT, E // _BLK - base_blk) - 1
    sems = {0: (ss0, sr0, se0, sw0), 1: (ss1, sr1, se1, sw1)}
    bufs = {0: (gs2.at[0], gr2.at[0], ep2.at[0], yb2.at[0]),
            1: (gs2.at[1], gr2.at[1], ep2.at[1], yb2.at[1])}

    pltpu.sync_copy(ge_hbm, gam)
    pltpu.sync_copy(be_hbm, bet)
    pltpu.sync_copy(snd_hbm.at[pl.ds(wid * _CH, _CH)], sidx_all)
    pltpu.sync_copy(rcv_hbm.at[pl.ds(wid * _CH, _CH)], ridx_all)

    def _gathers(p, b):
        bc = jnp.minimum(b, last)
        off = (base_blk + bc) * _BLK
        gs_b, gr_b, ep_b, _ = bufs[p]
        sm = sems[p]
        return (
            pltpu.make_async_copy(
                ps_hbm.at[sidx_all.at[pl.ds(bc * _BLK, _BLK)]], gs_b, sm[0]),
            pltpu.make_async_copy(
                pr_hbm.at[ridx_all.at[pl.ds(bc * _BLK, _BLK)]], gr_b, sm[1]),
            pltpu.make_async_copy(ep_hbm.at[pl.ds(off, _BLK)], ep_b, sm[2]),
        )

    def _wb(p, b):
        bc = jnp.minimum(b, last)
        off = (base_blk + bc) * _BLK
        return pltpu.make_async_copy(bufs[p][3], out_hbm.at[pl.ds(off, _BLK)],
                                     sems[p][3])

    def _start_g(p, b):
        for cp in _gathers(p, b):
            cp.start()

    def _wait_g(p, b):
        for cp in _gathers(p, b):
            cp.wait()

    def _compute(p):
        @plsc.parallel_loop(0, _BLK, 1, unroll=8)
        def _edge(e):
            ys = [ep2[p, e, pl.ds(_L * j, _L)] + gs2[p, e, pl.ds(_L * j, _L)]
                  + gr2[p, e, pl.ds(_L * j, _L)] for j in range(nv)]
            acc = ys[0]
            for j in range(1, nv):
                acc = acc + ys[j]
            s1 = jnp.sum(acc)
            acc2 = ys[0] * ys[0]
            for j in range(1, nv):
                acc2 = acc2 + ys[j] * ys[j]
            s2 = jnp.sum(acc2)
            mu = s1 * (1.0 / H)
            var = s2 * (1.0 / H) - mu * mu
            inv = _rsqrt_scalar(var + 1e-5)
            for j in range(nv):
                sl = pl.ds(_L * j, _L)
                o = (ys[j] - mu) * inv * gam[sl] + bet[sl]
                yb2[p, e, sl] = jnp.maximum(o, 0.0)

    nh = (last + 1) // 2
    _start_g(0, 0)
    _start_g(1, 1)
    _wait_g(0, 0)
    _compute(0)
    _wb(0, 0).start()
    _start_g(0, 2)
    _wait_g(1, 1)
    _compute(1)
    _wb(1, 1).start()
    _start_g(1, 3)

    @pl.loop(1, nh - 1)
    def _h(h):
        b0 = 2 * h
        b1 = b0 + 1
        _wait_g(0, b0)
        _wb(0, b0 - 2).wait()
        _compute(0)
        _wb(0, b0).start()
        _start_g(0, b0 + 2)
        _wait_g(1, b1)
        _wb(1, b1 - 2).wait()
        _compute(1)
        _wb(1, b1).start()
        _start_g(1, b1 + 2)

    b0 = 2 * (nh - 1)
    b1 = b0 + 1
    _wait_g(0, b0)
    _wb(0, b0 - 2).wait()
    _compute(0)
    _wb(0, b0).start()
    _wait_g(1, b1)
    _wb(1, b1 - 2).wait()
    _compute(1)
    _wb(1, b1).start()
    _wb(0, b0).wait()
    _wb(1, b1).wait()


def _agg_sc_body(rows_hbm, srcat_hbm, out_hbm, idx2, rows2, zb, acc,
                 si0, si1, sr0, sr1):
    E, H = rows_hbm.shape
    N = acc.shape[0]
    c = lax.axis_index("c")
    s = lax.axis_index("s")
    nzb = zb.shape[0]
    nzblk = N // nzb
    rounds = (nzblk + _NS - 1) // _NS

    @pl.loop(0, nzb)
    def _zrow(r):
        for j in range(H // _L):
            zb[r, pl.ds(_L * j, _L)] = jnp.zeros((_L,), jnp.float32)

    for q in range(rounds):
        g = s + q * _NS

        @pl.when(g < nzblk)
        def _():
            pltpu.sync_copy(zb, acc.at[pl.ds(g * nzb, nzb)])

    plsc.subcore_barrier()

    ept = E // _NS
    nblk = ept // _BLKC
    lastb = nblk - 1
    isems = {0: si0, 1: si1}
    rsems = {0: sr0, 1: sr1}

    def _in(p, b):
        bc = jnp.minimum(b, lastb)
        off = s * ept + bc * _BLKC
        return (
            pltpu.make_async_copy(srcat_hbm.at[pl.ds(c * E + off, _BLKC)],
                                  idx2.at[p], isems[p]),
            pltpu.make_async_copy(rows_hbm.at[pl.ds(off, _BLKC)],
                                  rows2.at[p], rsems[p]),
        )

    def _start_in(p, b):
        for cp in _in(p, b):
            cp.start()

    def _wait_in(p, b):
        for cp in _in(p, b):
            cp.wait()

    def _scatter(p):
        pltpu.sync_copy(rows2.at[p], acc.at[idx2.at[p]], add=True)

    _start_in(0, 0)
    _start_in(1, 1)

    @pl.loop(0, nblk // 2)
    def _h(h):
        b0 = 2 * h
        b1 = b0 + 1
        _wait_in(0, b0)
        _scatter(0)
        _start_in(0, b0 + 2)
        _wait_in(1, b1)
        _scatter(1)
        _start_in(1, b1 + 2)

    _wait_in(0, lastb)
    _scatter(0)
    _wait_in(1, lastb + 1)

    plsc.subcore_barrier()
    for q in range(rounds):
        g = s + q * _NS

        @pl.when(g < nzblk)
        def _():
            sl = pl.ds(g * nzb, nzb)
            pltpu.sync_copy(acc.at[sl], out_hbm.at[c, sl])


def kernel(nodes, edges, receivers, senders, globals_, n_node, n_edge,
           W_e, b_e, g_e, beta_e, W_n, b_n, g_n, beta_n):
    N, Dn = nodes.shape
    E, De = edges.shape
    Dg = globals_.shape[1]
    H = W_e.shape[1]

    W_e0 = W_e[:De]
    W_es = W_e[De:De + Dn]
    W_er = W_e[De + Dn:De + 2 * Dn]
    W_eg = W_e[De + 2 * Dn:]
    W_n1 = W_n[:Dn]
    W_n2 = W_n[Dn:Dn + H]
    W_n3 = W_n[Dn + H:Dn + 2 * H]
    W_ng = W_n[Dn + 2 * H:]

    snd = senders.astype(jnp.int32)
    rcv = receivers.astype(jnp.int32)
    srcat = jnp.concatenate([snd, rcv])
    pad = _NW * _CH - E
    snd_p = jnp.pad(snd, (0, pad))
    rcv_p = jnp.pad(rcv, (0, pad))

    bn = 1000
    be = 4000

    nproj = N // bn
    P_s, P_r, E_pre = pl.pallas_call(
        functools.partial(_prep_body, nproj),
        grid=(E // be,),
        in_specs=[
            pl.BlockSpec((bn, Dn), lambda i: (jnp.minimum(i, nproj - 1), 0)),
            pl.BlockSpec((Dn, H), lambda i: (0, 0)),
            pl.BlockSpec((Dn, H), lambda i: (0, 0)),
            pl.BlockSpec((be, De), lambda i: (i, 0)),
            pl.BlockSpec((De, H), lambda i: (0, 0)),
            pl.BlockSpec((1, Dg), lambda i: (0, 0)),
            pl.BlockSpec((Dg, H), lambda i: (0, 0)),
            pl.BlockSpec((1, H), lambda i: (0, 0)),
        ],
        out_specs=[
            pl.BlockSpec((bn, H), lambda i: (jnp.minimum(i, nproj - 1), 0)),
            pl.BlockSpec((bn, H), lambda i: (jnp.minimum(i, nproj - 1), 0)),
            pl.BlockSpec((be, H), lambda i: (i, 0)),
        ],
        out_shape=[
            jax.ShapeDtypeStruct((N, H), jnp.float32),
            jax.ShapeDtypeStruct((N, H), jnp.float32),
            jax.ShapeDtypeStruct((E, H), jnp.float32),
        ],
    )(nodes, W_es, W_er, edges, W_e0, globals_, W_eg, b_e.reshape(1, H))

    mesh = plsc.VectorSubcoreMesh(core_axis_name="c", subcore_axis_name="s")
    sc_params = pltpu.CompilerParams()
    if "needs_layout_passes" in getattr(
            pltpu.CompilerParams, "__dataclass_fields__", {}):
        sc_params = dataclasses.replace(sc_params, needs_layout_passes=False)

    edge_sc = pl.kernel(
        _edge_sc_body,
        out_type=jax.ShapeDtypeStruct((E, H), jnp.float32),
        mesh=mesh,
        scratch_types=[
            pltpu.VMEM((_CH,), jnp.int32),
            pltpu.VMEM((_CH,), jnp.int32),
            pltpu.VMEM((2, _BLK, H), jnp.float32),
            pltpu.VMEM((2, _BLK, H), jnp.float32),
            pltpu.VMEM((2, _BLK, H), jnp.float32),
            pltpu.VMEM((2, _BLK, H), jnp.float32),
            pltpu.VMEM((H,), jnp.float32),
            pltpu.VMEM((H,), jnp.float32),
        ] + [pltpu.SemaphoreType.DMA] * 8,
        compiler_params=sc_params,
    )
    new_edges = edge_sc(P_s, P_r, E_pre, snd_p, rcv_p, g_e, beta_e)

    agg_sc = pl.kernel(
        _agg_sc_body,
        out_type=jax.ShapeDtypeStruct((2, N, H), jnp.float32),
        mesh=mesh,
        scratch_types=[
            pltpu.VMEM((2, _BLKC), jnp.int32),
            pltpu.VMEM((2, _BLKC, H), jnp.float32),
            pltpu.VMEM((200, H), jnp.float32),
            pltpu.VMEM_SHARED((N, H), jnp.float32),
        ] + [pltpu.SemaphoreType.DMA] * 4,
    )
    aggs = agg_sc(new_edges, srcat)

    new_nodes = pl.pallas_call(
        _node_body,
        grid=(N // bn,),
        in_specs=[
            pl.BlockSpec((bn, Dn), lambda i: (i, 0)),
            pl.BlockSpec((1, bn, H), lambda i: (0, i, 0)),
            pl.BlockSpec((1, bn, H), lambda i: (1, i, 0)),
            pl.BlockSpec((Dn, H), lambda i: (0, 0)),
            pl.BlockSpec((H, H), lambda i: (0, 0)),
            pl.BlockSpec((H, H), lambda i: (0, 0)),
            pl.BlockSpec((1, Dg), lambda i: (0, 0)),
            pl.BlockSpec((Dg, H), lambda i: (0, 0)),
            pl.BlockSpec((1, H), lambda i: (0, 0)),
            pl.BlockSpec((1, H), lambda i: (0, 0)),
            pl.BlockSpec((1, H), lambda i: (0, 0)),
        ],
        out_specs=pl.BlockSpec((bn, H), lambda i: (i, 0)),
        out_shape=jax.ShapeDtypeStruct((N, H), jnp.float32),
    )(nodes, aggs, aggs, W_n1, W_n2, W_n3, globals_, W_ng,
      b_n.reshape(1, H), g_n.reshape(1, H), beta_n.reshape(1, H))

    return (new_nodes, new_edges)
